# baseline (speedup 1.0000x reference)
import functools

import jax
import jax.numpy as jnp
from jax import lax
from jax.experimental import pallas as pl
from jax.experimental.pallas import tpu as pltpu
from jax.experimental.pallas import tpu_sc as plsc

NUM_REL = 3
DROP_P = 0.3

_NCORES = 2
_NSUB = 16
_CW = 16

_TC_PARAMS = pltpu.CompilerParams(vmem_limit_bytes=128 * 1024 * 1024)
_SC_PARAMS = pltpu.CompilerParams(use_tc_tiling_on_sc=False)


def _acc_rows(half):
    return -(-(NUM_REL * half + 1) // 16) * 16


def _tc_idx_body(src_ref, dst_ref, typ_ref, gidx_ref, ridx_ref, ridxf_ref,
                 *, n_nodes):
    sv = src_ref[...]
    dv = dst_ref[...]
    tv = typ_ref[...]
    half = n_nodes // _NCORES
    trash = NUM_REL * half
    gidx_ref[...] = tv * n_nodes + sv
    ridxf_ref[...] = tv * n_nodes + dv
    for c in range(_NCORES):
        base = c * half
        owned = (dv >= base) & (dv < base + half)
        ridx_ref[c] = jnp.where(owned, tv * half + (dv - base), trash)


def _transforms(h, w_ref, s_ref, y_ref):
    dn = (((1,), (1,)), ((), ()))
    s_ref[...] = lax.dot_general(h, w_ref[0], dn,
                                 preferred_element_type=jnp.float32)
    for r in range(NUM_REL):
        y_ref[r] = lax.dot_general(h, w_ref[r + 1], dn,
                                   preferred_element_type=jnp.float32)


def _tc1_body(x_ref, w_ref, s_ref, y_ref):
    _transforms(x_ref[...], w_ref, s_ref, y_ref)


def _mean_agg(dump, cntd, h_dim, half, edge_split):
    if edge_split:
        n = half * _NCORES
        acc = None
        for r in range(NUM_REL):
            blk = dump[0, r * n:(r + 1) * n, :] + dump[1, r * n:(r + 1) * n, :]
            cnt = jnp.concatenate(
                [cntd[c, r * half:(r + 1) * half, 0:1]
                 for c in range(_NCORES)], axis=0)
            term = blk / jnp.maximum(cnt, 1.0)
            acc = term if acc is None else acc + term
        return acc
    parts = []
    for c in range(_NCORES):
        acc = None
        for r in range(NUM_REL):
            blk = dump[c, r * half:(r + 1) * half, :]
            cnt = cntd[c, r * half:(r + 1) * half, 0:1]
            term = blk / jnp.maximum(cnt, 1.0)
            acc = term if acc is None else acc + term
        parts.append(acc)
    return jnp.concatenate(parts, axis=0)


def _bn_relu_drop(h, g, b, m):
    mu = jnp.mean(h, axis=0, keepdims=True)
    d = h - mu
    var = jnp.mean(d * d, axis=0, keepdims=True)
    h = d * lax.rsqrt(var + 1e-5) * g[None, :] + b[None, :]
    return jnp.maximum(h, 0.0) * m


def _tc_norm_body(s_ref, dump_ref, cnt_ref, b_ref, g_ref, bb_ref, m_ref,
                  h_ref):
    h_dim = s_ref.shape[1]
    half = s_ref.shape[0] // _NCORES
    agg = _mean_agg(dump_ref[...], cnt_ref[...], h_dim, half, False)
    h = s_ref[...] + agg + b_ref[...][None, :]
    h_ref[...] = _bn_relu_drop(h, g_ref[...], bb_ref[...], m_ref[...])


def _tc_sum2_body(d_ref, o_ref):
    o_ref[...] = d_ref[0] + d_ref[1]


def _tc_norm2_body(s_ref, dump_ref, cnt_ref, b_ref, g_ref, bb_ref, m_ref,
                   h_ref):
    half = s_ref.shape[0] // _NCORES
    n = s_ref.shape[0]
    dump = dump_ref[...]
    agg = None
    for r in range(NUM_REL):
        blk = dump[r * n:(r + 1) * n, :]
        cnt = jnp.concatenate(
            [cnt_ref[c, r * half:(r + 1) * half, 0:1]
             for c in range(_NCORES)], axis=0)
        term = blk / jnp.maximum(cnt, 1.0)
        agg = term if agg is None else agg + term
    h = s_ref[...] + agg + b_ref[...][None, :]
    h_ref[...] = _bn_relu_drop(h, g_ref[...], bb_ref[...], m_ref[...])


def _tc_head_body(h_ref, w1_ref, b1_ref, m3_ref, w2_ref, b2_ref, out_ref):
    h = h_ref[...]
    dn = (((1,), (1,)), ((), ()))
    h = lax.dot_general(h, w1_ref[...], dn,
                        preferred_element_type=jnp.float32) \
        + b1_ref[...][None, :]
    h = jnp.maximum(h, 0.0) * m3_ref[...]
    out_ref[...] = lax.dot_general(h, w2_ref[...], dn,
                                   preferred_element_type=jnp.float32) \
        + b2_ref[...][None, :]


def _zero_rows16(buf):
    z = jnp.zeros((16,), jnp.float32)
    w = buf.shape[1]
    for i in range(16):
        for k in range(w // 16):
            buf[i, pl.ds(k * 16, 16)] = z
        if w % 16:
            buf[i, pl.ds(w - 16, 16)] = z


def _coverage_start(s, acc_rows):
    quota = -(-acc_rows // _NSUB // 16) * 16
    return jnp.minimum(s * quota, acc_rows - quota), quota // 16


def _make_sc_counts(n_nodes, n_edges):
    half = n_nodes // _NCORES
    acc_rows = _acc_rows(half)
    chunk = 128
    ept = n_edges // _NSUB
    nchunks = ept // chunk
    tail = ept - nchunks * chunk
    assert ept * _NSUB == n_edges and tail % 8 == 0

    mesh = plsc.VectorSubcoreMesh(core_axis_name="c", subcore_axis_name="s")

    @functools.partial(
        pl.kernel,
        out_type=jax.ShapeDtypeStruct((_NCORES, acc_rows, _CW), jnp.float32),
        mesh=mesh,
        compiler_params=_SC_PARAMS,
        scratch_types=[
            pltpu.VMEM_SHARED((acc_rows, _CW), jnp.float32),
            pltpu.VMEM((chunk, _CW), jnp.float32),
            pltpu.VMEM((16, _CW), jnp.float32),
            pltpu.VMEM((chunk,), jnp.int32),
            pltpu.VMEM((max(tail, 8),), jnp.int32),
        ],
    )
    def sc_counts(ridx_hbm, out_hbm, cnt, ones, zb, rbuf, rtail):
        c = lax.axis_index("c")
        s = lax.axis_index("s")
        zstart, ncopies = _coverage_start(s, acc_rows)

        one16 = jnp.full((16,), 1.0, jnp.float32)
        for i in range(chunk):
            ones[i, pl.ds(0, 16)] = one16
        _zero_rows16(zb)

        def zloop(z, carry):
            pltpu.sync_copy(zb, cnt.at[pl.ds(zstart + z * 16, 16)])
            return carry

        lax.fori_loop(0, ncopies, zloop, 0)
        plsc.subcore_barrier()

        def body(k, carry):
            e0 = s * ept + k * chunk
            pltpu.sync_copy(ridx_hbm.at[c, pl.ds(e0, chunk)], rbuf)
            pltpu.sync_copy(ones, cnt.at[rbuf], add=True)
            return carry

        lax.fori_loop(0, nchunks, body, 0)
        if tail:
            e0 = s * ept + nchunks * chunk
            pltpu.sync_copy(ridx_hbm.at[c, pl.ds(e0, tail)], rtail)
            pltpu.sync_copy(ones.at[pl.ds(0, tail)], cnt.at[rtail], add=True)
        plsc.subcore_barrier()

        def dump(z, carry):
            o = zstart + z * 16
            pltpu.sync_copy(cnt.at[pl.ds(o, 16)], zb)
            pltpu.sync_copy(zb, out_hbm.at[c, pl.ds(o, 16)])
            return carry

        lax.fori_loop(0, ncopies, dump, 0)

    return sc_counts


def _make_sc_agg(n_nodes, n_edges, h_dim, chunk, nbuf, edge_split):
    half = n_nodes // _NCORES
    if edge_split:
        acc_rows = -(-(NUM_REL * n_nodes) // 16) * 16
        ept = n_edges // (_NCORES * _NSUB)
    else:
        acc_rows = _acc_rows(half)
        ept = n_edges // _NSUB
    nchunks = ept // chunk
    assert nchunks * chunk == ept and ept * _NSUB * (
        _NCORES if edge_split else 1) == n_edges
    assert chunk % 8 == 0 and chunk <= 128
    assert nbuf in (1, 2)
    npairs = nchunks // 2

    mesh = plsc.VectorSubcoreMesh(core_axis_name="c", subcore_axis_name="s")

    scratch = [pltpu.VMEM_SHARED((acc_rows, h_dim), jnp.float32)]
    scratch += [pltpu.VMEM((chunk, h_dim), jnp.float32)] * nbuf
    scratch += [pltpu.VMEM((chunk,), jnp.int32)] * nbuf
    scratch += [pltpu.VMEM((chunk,), jnp.int32)] * nbuf
    scratch += [pltpu.SemaphoreType.DMA] * nbuf

    @functools.partial(
        pl.kernel,
        out_type=jax.ShapeDtypeStruct((_NCORES, acc_rows, h_dim), jnp.float32),
        mesh=mesh,
        compiler_params=_SC_PARAMS,
        scratch_types=scratch,
    )
    def sc_agg(y_hbm, gidx_hbm, ridx_hbm, out_hbm, acc, *bufs):
        rows = bufs[:nbuf]
        gbuf = bufs[nbuf:2 * nbuf]
        rbuf = bufs[2 * nbuf:3 * nbuf]
        sems = bufs[3 * nbuf:]
        c = lax.axis_index("c")
        s = lax.axis_index("s")
        zstart, ncopies = _coverage_start(s, acc_rows)

        _zero_rows16(rows[0])

        def zloop(z, carry):
            pltpu.sync_copy(rows[0].at[pl.ds(0, 16)],
                            acc.at[pl.ds(zstart + z * 16, 16)])
            return carry

        lax.fori_loop(0, ncopies, zloop, 0)
        plsc.subcore_barrier()

        def stage(k, b):
            if edge_split:
                e0 = c * (n_edges // _NCORES) + s * ept + k * chunk
                pltpu.sync_copy(ridx_hbm.at[pl.ds(e0, chunk)], rbuf[b])
            else:
                e0 = s * ept + k * chunk
                pltpu.sync_copy(ridx_hbm.at[c, pl.ds(e0, chunk)], rbuf[b])
            pltpu.sync_copy(gidx_hbm.at[pl.ds(e0, chunk)], gbuf[b])

        def fire(b):
            pltpu.async_copy(y_hbm.at[gbuf[b]], rows[b], sems[b])

        def wait(b):
            pltpu.make_async_copy(y_hbm.at[gbuf[b]], rows[b], sems[b]).wait()

        def scatter(b):
            pltpu.sync_copy(rows[b], acc.at[rbuf[b]], add=True)

        if nbuf == 1:
            def body(k, carry):
                stage(k, 0)
                fire(0)
                wait(0)
                scatter(0)
                return carry

            lax.fori_loop(0, nchunks, body, 0)
        else:
            stage(0, 0)
            fire(0)

            def pair(m, carry):
                k0 = 2 * m
                wait(0)
                stage(k0 + 1, 1)
                fire(1)
                scatter(0)
                wait(1)

                @pl.when(m < npairs - 1)
                def _():
                    stage(k0 + 2, 0)
                    fire(0)

                scatter(1)
                return carry

            lax.fori_loop(0, npairs, pair, 0)
            if nchunks % 2:
                stage(nchunks - 1, 0)
                fire(0)
                wait(0)
                scatter(0)
        plsc.subcore_barrier()

        def dump(z, carry):
            o = zstart + z * 16
            pltpu.sync_copy(acc.at[pl.ds(o, 16)], rows[0].at[pl.ds(0, 16)])
            pltpu.sync_copy(rows[0].at[pl.ds(0, 16)],
                            out_hbm.at[c, pl.ds(o, 16)])
            return carry

        lax.fori_loop(0, ncopies, dump, 0)

    return sc_agg


def kernel(x, edge_index, edge_type, W_self1, W_rel1, bias1,
           W_self2, W_rel2, bias2, bn1_g, bn1_b, bn2_g, bn2_b,
           cls_w1, cls_b1, cls_w2, cls_b2):
    n, _ = x.shape
    e = edge_index.shape[1]
    h1 = W_self1.shape[0]
    h2 = W_self2.shape[0]
    c1 = cls_w1.shape[0]

    dk = jax.random.key(1234)
    k1, k2, k3 = jax.random.split(dk, 3)
    scale = 1.0 / (1.0 - DROP_P)
    m1 = jax.random.bernoulli(k1, 1.0 - DROP_P, (n, h1)).astype(jnp.float32) * scale
    m2 = jax.random.bernoulli(k2, 1.0 - DROP_P, (n, h2)).astype(jnp.float32) * scale
    m3 = jax.random.bernoulli(k3, 1.0 - DROP_P, (n, c1)).astype(jnp.float32) * scale

    wcat1 = jnp.concatenate([W_self1[None], W_rel1], axis=0)
    wcat2 = jnp.concatenate([W_self2[None], W_rel2], axis=0)

    ecols = 512
    erows = e // ecols
    gidx2, ridx2, ridxf2 = pl.pallas_call(
        functools.partial(_tc_idx_body, n_nodes=n),
        out_shape=[
            jax.ShapeDtypeStruct((erows, ecols), jnp.int32),
            jax.ShapeDtypeStruct((_NCORES, erows, ecols), jnp.int32),
            jax.ShapeDtypeStruct((erows, ecols), jnp.int32),
        ],
        compiler_params=_TC_PARAMS,
    )(edge_index[0].reshape(erows, ecols),
      edge_index[1].reshape(erows, ecols),
      edge_type.reshape(erows, ecols))
    gidx = gidx2.reshape(e)
    ridx = ridx2.reshape(_NCORES, e)
    ridxf = ridxf2.reshape(e)

    cntd = _make_sc_counts(n, e)(ridx)

    s1, y1 = pl.pallas_call(
        _tc1_body,
        out_shape=[
            jax.ShapeDtypeStruct((n, h1), jnp.float32),
            jax.ShapeDtypeStruct((NUM_REL, n, h1), jnp.float32),
        ],
        compiler_params=_TC_PARAMS,
    )(x, wcat1)
    dump1 = _make_sc_agg(n, e, h1, 40, 2, False)(y1.reshape(NUM_REL * n, h1),
                                                 gidx, ridx)

    h1v = pl.pallas_call(
        _tc_norm_body,
        out_shape=jax.ShapeDtypeStruct((n, h1), jnp.float32),
        compiler_params=_TC_PARAMS,
    )(s1, dump1, cntd, bias1, bn1_g, bn1_b, m1)
    s2, y2 = pl.pallas_call(
        _tc1_body,
        out_shape=[
            jax.ShapeDtypeStruct((n, h2), jnp.float32),
            jax.ShapeDtypeStruct((NUM_REL, n, h2), jnp.float32),
        ],
        compiler_params=_TC_PARAMS,
    )(h1v, wcat2)
    dump2 = _make_sc_agg(n, e, h2, 80, 2, True)(y2.reshape(NUM_REL * n, h2),
                                                gidx, ridxf)

    dsum2 = pl.pallas_call(
        _tc_sum2_body,
        out_shape=jax.ShapeDtypeStruct(dump2.shape[1:], jnp.float32),
        compiler_params=_TC_PARAMS,
    )(dump2)
    h2v = pl.pallas_call(
        _tc_norm2_body,
        out_shape=jax.ShapeDtypeStruct((n, h2), jnp.float32),
        compiler_params=_TC_PARAMS,
    )(s2, dsum2, cntd, bias2, bn2_g, bn2_b, m2)
    out = pl.pallas_call(
        _tc_head_body,
        out_shape=jax.ShapeDtypeStruct((n, cls_w2.shape[0]), jnp.float32),
        compiler_params=_TC_PARAMS,
    )(h2v, cls_w1, cls_b1, m3, cls_w2, cls_b2)
    return out


# speedup vs baseline: 1.2292x; 1.2292x over previous
import functools

import jax
import jax.numpy as jnp
from jax import lax
from jax.experimental import pallas as pl
from jax.experimental.pallas import tpu as pltpu
from jax.experimental.pallas import tpu_sc as plsc

NUM_REL = 3
DROP_P = 0.3

_NCORES = 2
_NSUB = 16
_CW = 16

_TC_PARAMS = pltpu.CompilerParams(vmem_limit_bytes=128 * 1024 * 1024)
_SC_PARAMS = pltpu.CompilerParams(use_tc_tiling_on_sc=False)


def _acc_rows(half):
    return -(-(NUM_REL * half + 1) // 16) * 16


def _tc_idx_body(src_ref, dst_ref, typ_ref, gidx_ref, ridx_ref, ridxf_ref,
                 *, n_nodes):
    sv = src_ref[...]
    dv = dst_ref[...]
    tv = typ_ref[...]
    half = n_nodes // _NCORES
    trash = NUM_REL * half
    gidx_ref[...] = tv * n_nodes + sv
    ridxf_ref[...] = tv * n_nodes + dv
    for c in range(_NCORES):
        base = c * half
        owned = (dv >= base) & (dv < base + half)
        ridx_ref[c] = jnp.where(owned, tv * half + (dv - base), trash)


def _transforms(h, w_ref, s_ref, y_ref):
    dn = (((1,), (1,)), ((), ()))
    s_ref[...] = lax.dot_general(h, w_ref[0], dn,
                                 preferred_element_type=jnp.float32)
    for r in range(NUM_REL):
        y_ref[r] = lax.dot_general(h, w_ref[r + 1], dn,
                                   preferred_element_type=jnp.float32)


def _tc1_body(x_ref, w_ref, s_ref, y_ref):
    _transforms(x_ref[...], w_ref, s_ref, y_ref)


def _mean_agg(dump, cntd, h_dim, half, edge_split):
    if edge_split:
        n = half * _NCORES
        acc = None
        for r in range(NUM_REL):
            blk = dump[0, r * n:(r + 1) * n, :] + dump[1, r * n:(r + 1) * n, :]
            cnt = jnp.concatenate(
                [cntd[c, r * half:(r + 1) * half, 0:1]
                 for c in range(_NCORES)], axis=0)
            term = blk / jnp.maximum(cnt, 1.0)
            acc = term if acc is None else acc + term
        return acc
    parts = []
    for c in range(_NCORES):
        acc = None
        for r in range(NUM_REL):
            blk = dump[c, r * half:(r + 1) * half, :]
            cnt = cntd[c, r * half:(r + 1) * half, 0:1]
            term = blk / jnp.maximum(cnt, 1.0)
            acc = term if acc is None else acc + term
        parts.append(acc)
    return jnp.concatenate(parts, axis=0)


def _bn_relu_drop(h, g, b, m):
    mu = jnp.mean(h, axis=0, keepdims=True)
    d = h - mu
    var = jnp.mean(d * d, axis=0, keepdims=True)
    h = d * lax.rsqrt(var + 1e-5) * g[None, :] + b[None, :]
    return jnp.maximum(h, 0.0) * m


def _tc_norm_body(s_ref, dump_ref, cnt_ref, b_ref, g_ref, bb_ref, m_ref,
                  h_ref):
    h_dim = s_ref.shape[1]
    half = s_ref.shape[0] // _NCORES
    agg = _mean_agg(dump_ref[...], cnt_ref[...], h_dim, half, False)
    h = s_ref[...] + agg + b_ref[...][None, :]
    h_ref[...] = _bn_relu_drop(h, g_ref[...], bb_ref[...], m_ref[...])


def _tc_sum2_body(d_ref, o_ref):
    o_ref[...] = d_ref[0] + d_ref[1]


def _tc_norm2_body(s_ref, dump_ref, cnt_ref, b_ref, g_ref, bb_ref, m_ref,
                   h_ref):
    half = s_ref.shape[0] // _NCORES
    n = s_ref.shape[0]
    dump = dump_ref[...]
    agg = None
    for r in range(NUM_REL):
        blk = dump[r * n:(r + 1) * n, :]
        cnt = jnp.concatenate(
            [cnt_ref[c, r * half:(r + 1) * half, 0:1]
             for c in range(_NCORES)], axis=0)
        term = blk / jnp.maximum(cnt, 1.0)
        agg = term if agg is None else agg + term
    h = s_ref[...] + agg + b_ref[...][None, :]
    h_ref[...] = _bn_relu_drop(h, g_ref[...], bb_ref[...], m_ref[...])


def _tc_head_body(h_ref, w1_ref, b1_ref, m3_ref, w2_ref, b2_ref, out_ref):
    h = h_ref[...]
    dn = (((1,), (1,)), ((), ()))
    h = lax.dot_general(h, w1_ref[...], dn,
                        preferred_element_type=jnp.float32) \
        + b1_ref[...][None, :]
    h = jnp.maximum(h, 0.0) * m3_ref[...]
    out_ref[...] = lax.dot_general(h, w2_ref[...], dn,
                                   preferred_element_type=jnp.float32) \
        + b2_ref[...][None, :]


def _zero_rows16(buf):
    z = jnp.zeros((16,), jnp.float32)
    w = buf.shape[1]
    for i in range(16):
        for k in range(w // 16):
            buf[i, pl.ds(k * 16, 16)] = z
        if w % 16:
            buf[i, pl.ds(w - 16, 16)] = z


def _coverage_start(s, acc_rows):
    quota = -(-acc_rows // _NSUB // 16) * 16
    return jnp.minimum(s * quota, acc_rows - quota), quota // 16


def _zero_rows_all(buf):
    z = jnp.zeros((16,), jnp.float32)
    w = buf.shape[1]
    for i in range(buf.shape[0]):
        for k in range(w // 16):
            buf[i, pl.ds(k * 16, 16)] = z
        if w % 16:
            buf[i, pl.ds(w - 16, 16)] = z


def _make_sc_counts(n_nodes, n_edges):
    half = n_nodes // _NCORES
    acc_rows = _acc_rows(half)
    chunk = 128
    ept = n_edges // _NSUB
    nchunks = ept // chunk
    tail = ept - nchunks * chunk
    assert ept * _NSUB == n_edges and tail % 8 == 0

    mesh = plsc.VectorSubcoreMesh(core_axis_name="c", subcore_axis_name="s")

    @functools.partial(
        pl.kernel,
        out_type=jax.ShapeDtypeStruct((_NCORES, acc_rows, _CW), jnp.float32),
        mesh=mesh,
        compiler_params=_SC_PARAMS,
        scratch_types=[
            pltpu.VMEM_SHARED((acc_rows, _CW), jnp.float32),
            pltpu.VMEM((chunk, _CW), jnp.float32),
            pltpu.VMEM((16, _CW), jnp.float32),
            pltpu.VMEM((chunk,), jnp.int32),
            pltpu.VMEM((max(tail, 8),), jnp.int32),
        ],
    )
    def sc_counts(ridx_hbm, out_hbm, cnt, ones, zb, rbuf, rtail):
        c = lax.axis_index("c")
        s = lax.axis_index("s")
        zstart, ncopies = _coverage_start(s, acc_rows)

        one16 = jnp.full((16,), 1.0, jnp.float32)
        for i in range(chunk):
            ones[i, pl.ds(0, 16)] = one16
        _zero_rows16(zb)

        def zloop(z, carry):
            pltpu.sync_copy(zb, cnt.at[pl.ds(zstart + z * 16, 16)])
            return carry

        lax.fori_loop(0, ncopies, zloop, 0)
        plsc.subcore_barrier()

        def body(k, carry):
            e0 = s * ept + k * chunk
            pltpu.sync_copy(ridx_hbm.at[c, pl.ds(e0, chunk)], rbuf)
            pltpu.sync_copy(ones, cnt.at[rbuf], add=True)
            return carry

        lax.fori_loop(0, nchunks, body, 0)
        if tail:
            e0 = s * ept + nchunks * chunk
            pltpu.sync_copy(ridx_hbm.at[c, pl.ds(e0, tail)], rtail)
            pltpu.sync_copy(ones.at[pl.ds(0, tail)], cnt.at[rtail], add=True)
        plsc.subcore_barrier()

        def dump(z, carry):
            o = zstart + z * 16
            pltpu.sync_copy(cnt.at[pl.ds(o, 16)], zb)
            pltpu.sync_copy(zb, out_hbm.at[c, pl.ds(o, 16)])
            return carry

        lax.fori_loop(0, ncopies, dump, 0)

    return sc_counts


def _make_sc_agg(n_nodes, n_edges, h_dim, chunk, nbuf, edge_split):
    half = n_nodes // _NCORES
    if edge_split:
        acc_rows = -(-(NUM_REL * n_nodes) // 16) * 16
        ept = n_edges // (_NCORES * _NSUB)
    else:
        acc_rows = _acc_rows(half)
        ept = n_edges // _NSUB
    nchunks = ept // chunk
    assert nchunks * chunk == ept and ept * _NSUB * (
        _NCORES if edge_split else 1) == n_edges
    assert chunk % 8 == 0 and chunk <= 128
    assert nbuf in (1, 2)
    npairs = nchunks // 2

    mesh = plsc.VectorSubcoreMesh(core_axis_name="c", subcore_axis_name="s")

    scratch = [pltpu.VMEM_SHARED((acc_rows, h_dim), jnp.float32)]
    scratch += [pltpu.VMEM((chunk, h_dim), jnp.float32)] * nbuf
    scratch += [pltpu.VMEM((chunk,), jnp.int32)] * nbuf
    scratch += [pltpu.VMEM((chunk,), jnp.int32)] * nbuf
    scratch += [pltpu.SemaphoreType.DMA] * nbuf

    @functools.partial(
        pl.kernel,
        out_type=jax.ShapeDtypeStruct((_NCORES, acc_rows, h_dim), jnp.float32),
        mesh=mesh,
        compiler_params=_SC_PARAMS,
        scratch_types=scratch,
    )
    def sc_agg(y_hbm, gidx_hbm, ridx_hbm, out_hbm, acc, *bufs):
        rows = bufs[:nbuf]
        gbuf = bufs[nbuf:2 * nbuf]
        rbuf = bufs[2 * nbuf:3 * nbuf]
        sems = bufs[3 * nbuf:]
        c = lax.axis_index("c")
        s = lax.axis_index("s")
        zstart, _ = _coverage_start(s, acc_rows)
        quota = -(-acc_rows // _NSUB // 16) * 16
        ncb = -(-quota // chunk)

        _zero_rows_all(rows[0])

        def zloop(z, carry):
            o = zstart + jnp.minimum(z * chunk, quota - chunk)
            pltpu.sync_copy(rows[0], acc.at[pl.ds(o, chunk)])
            return carry

        lax.fori_loop(0, ncb, zloop, 0)
        plsc.subcore_barrier()

        def stage(k, b):
            if edge_split:
                e0 = c * (n_edges // _NCORES) + s * ept + k * chunk
                pltpu.sync_copy(ridx_hbm.at[pl.ds(e0, chunk)], rbuf[b])
            else:
                e0 = s * ept + k * chunk
                pltpu.sync_copy(ridx_hbm.at[c, pl.ds(e0, chunk)], rbuf[b])
            pltpu.sync_copy(gidx_hbm.at[pl.ds(e0, chunk)], gbuf[b])

        def fire(b):
            pltpu.async_copy(y_hbm.at[gbuf[b]], rows[b], sems[b])

        def wait(b):
            pltpu.make_async_copy(y_hbm.at[gbuf[b]], rows[b], sems[b]).wait()

        def scatter(b):
            pltpu.sync_copy(rows[b], acc.at[rbuf[b]], add=True)

        if nbuf == 1:
            def body(k, carry):
                stage(k, 0)
                fire(0)
                wait(0)
                scatter(0)
                return carry

            lax.fori_loop(0, nchunks, body, 0)
        else:
            stage(0, 0)
            fire(0)

            def pair(m, carry):
                k0 = 2 * m
                wait(0)
                stage(k0 + 1, 1)
                fire(1)
                scatter(0)
                wait(1)

                @pl.when(m < npairs - 1)
                def _():
                    stage(k0 + 2, 0)
                    fire(0)

                scatter(1)
                return carry

            lax.fori_loop(0, npairs, pair, 0)
            if nchunks % 2:
                stage(nchunks - 1, 0)
                fire(0)
                wait(0)
                scatter(0)
        plsc.subcore_barrier()

        def dump(z, carry):
            o = zstart + jnp.minimum(z * chunk, quota - chunk)
            pltpu.sync_copy(acc.at[pl.ds(o, chunk)], rows[0])
            pltpu.sync_copy(rows[0], out_hbm.at[c, pl.ds(o, chunk)])
            return carry

        lax.fori_loop(0, ncb, dump, 0)

    return sc_agg


def kernel(x, edge_index, edge_type, W_self1, W_rel1, bias1,
           W_self2, W_rel2, bias2, bn1_g, bn1_b, bn2_g, bn2_b,
           cls_w1, cls_b1, cls_w2, cls_b2):
    n, _ = x.shape
    e = edge_index.shape[1]
    h1 = W_self1.shape[0]
    h2 = W_self2.shape[0]
    c1 = cls_w1.shape[0]

    dk = jax.random.key(1234)
    k1, k2, k3 = jax.random.split(dk, 3)
    scale = 1.0 / (1.0 - DROP_P)
    m1 = jax.random.bernoulli(k1, 1.0 - DROP_P, (n, h1)).astype(jnp.float32) * scale
    m2 = jax.random.bernoulli(k2, 1.0 - DROP_P, (n, h2)).astype(jnp.float32) * scale
    m3 = jax.random.bernoulli(k3, 1.0 - DROP_P, (n, c1)).astype(jnp.float32) * scale

    wcat1 = jnp.concatenate([W_self1[None], W_rel1], axis=0)
    wcat2 = jnp.concatenate([W_self2[None], W_rel2], axis=0)

    ecols = 512
    erows = e // ecols
    gidx2, ridx2, ridxf2 = pl.pallas_call(
        functools.partial(_tc_idx_body, n_nodes=n),
        out_shape=[
            jax.ShapeDtypeStruct((erows, ecols), jnp.int32),
            jax.ShapeDtypeStruct((_NCORES, erows, ecols), jnp.int32),
            jax.ShapeDtypeStruct((erows, ecols), jnp.int32),
        ],
        compiler_params=_TC_PARAMS,
    )(edge_index[0].reshape(erows, ecols),
      edge_index[1].reshape(erows, ecols),
      edge_type.reshape(erows, ecols))
    gidx = gidx2.reshape(e)
    ridx = ridx2.reshape(_NCORES, e)
    ridxf = ridxf2.reshape(e)

    cntd = _make_sc_counts(n, e)(ridx)

    s1, y1 = pl.pallas_call(
        _tc1_body,
        out_shape=[
            jax.ShapeDtypeStruct((n, h1), jnp.float32),
            jax.ShapeDtypeStruct((NUM_REL, n, h1), jnp.float32),
        ],
        compiler_params=_TC_PARAMS,
    )(x, wcat1)
    dump1 = _make_sc_agg(n, e, h1, 80, 1, False)(y1.reshape(NUM_REL * n, h1),
                                                 gidx, ridx)

    h1v = pl.pallas_call(
        _tc_norm_body,
        out_shape=jax.ShapeDtypeStruct((n, h1), jnp.float32),
        compiler_params=_TC_PARAMS,
    )(s1, dump1, cntd, bias1, bn1_g, bn1_b, m1)
    s2, y2 = pl.pallas_call(
        _tc1_body,
        out_shape=[
            jax.ShapeDtypeStruct((n, h2), jnp.float32),
            jax.ShapeDtypeStruct((NUM_REL, n, h2), jnp.float32),
        ],
        compiler_params=_TC_PARAMS,
    )(h1v, wcat2)
    dump2 = _make_sc_agg(n, e, h2, 80, 2, True)(y2.reshape(NUM_REL * n, h2),
                                                gidx, ridxf)

    dsum2 = pl.pallas_call(
        _tc_sum2_body,
        out_shape=jax.ShapeDtypeStruct(dump2.shape[1:], jnp.float32),
        compiler_params=_TC_PARAMS,
    )(dump2)
    h2v = pl.pallas_call(
        _tc_norm2_body,
        out_shape=jax.ShapeDtypeStruct((n, h2), jnp.float32),
        compiler_params=_TC_PARAMS,
    )(s2, dsum2, cntd, bias2, bn2_g, bn2_b, m2)
    out = pl.pallas_call(
        _tc_head_body,
        out_shape=jax.ShapeDtypeStruct((n, cls_w2.shape[0]), jnp.float32),
        compiler_params=_TC_PARAMS,
    )(h2v, cls_w1, cls_b1, m3, cls_w2, cls_b2)
    return out
